# Initial kernel scaffold; baseline (speedup 1.0000x reference)
import jax
import jax.numpy as jnp
from jax.experimental import pallas as pl


def kernel(features, edge_vals, W1, b1, W2, b2, gamma, beta, edge_src, edge_dst):
    raise NotImplementedError("write your pallas kernel here")


# speedup vs baseline: 4.7350x; 4.7350x over previous
import functools

import jax
import jax.numpy as jnp
from jax import lax
import jax.experimental.pallas as pl
from jax.experimental.pallas import tpu as pltpu
from jax.experimental.pallas import tpu_sc as plsc

N = 10000
E = 320000
D = 128
G = 128
NG = E // G
NW = 32
RPT = 624

_mesh = plsc.VectorSubcoreMesh(core_axis_name="c", subcore_axis_name="s")


@functools.partial(
    pl.kernel,
    out_type=jax.ShapeDtypeStruct((2, N, D), jnp.float32),
    mesh=_mesh,
    scratch_types=[
        pltpu.VMEM((1, G), jnp.int32),
        pltpu.VMEM((1, G), jnp.int32),
        pltpu.VMEM((G,), jnp.float32),
        pltpu.VMEM((G, D), jnp.float32),
        pltpu.VMEM_SHARED((N, D), jnp.float32),
        pltpu.SemaphoreType.DMA,
    ],
    compiler_params=pltpu.CompilerParams(needs_layout_passes=False),
)
def _sc_scatter(feat_h, src_h, dst_h, vals_h, out_h,
                src_v, dst_v, vals_v, rows_v, acc, sem):
    c = lax.axis_index("c")
    s = lax.axis_index("s")
    w = c * 16 + s

    zv = jnp.zeros((16,), jnp.float32)

    def zrow(r, carry):
        for j in range(8):
            rows_v[r, pl.ds(j * 16, 16)] = zv
        return carry

    lax.fori_loop(0, G, zrow, 0)
    r0 = s * RPT
    for i in range(4):
        pltpu.sync_copy(rows_v, acc.at[pl.ds(r0 + i * G, G)])
    pltpu.sync_copy(rows_v.at[pl.ds(0, RPT - 4 * G)],
                    acc.at[pl.ds(r0 + 4 * G, RPT - 4 * G)])

    @pl.when(s == 15)
    def _():
        pltpu.sync_copy(rows_v.at[pl.ds(0, 16)], acc.at[pl.ds(15 * RPT + RPT, 16)])

    plsc.subcore_barrier()

    base = w * (NG // NW) + jnp.minimum(w, NG % NW)
    cnt = NG // NW + jnp.where(w < NG % NW, 1, 0)

    def group_body(i, carry):
        gg = base + i
        pltpu.sync_copy(src_h.at[pl.ds(gg, 1)], src_v)
        pltpu.sync_copy(dst_h.at[pl.ds(gg, 1)], dst_v)
        pltpu.sync_copy(vals_h.at[pl.ds(gg * G, G)], vals_v)
        pltpu.async_copy(feat_h.at[src_v.at[0]], rows_v, sem).wait()

        def edge_body(e, c2):
            vv = plsc.load_gather(vals_v, [jnp.full((16,), e, jnp.int32)])
            for j in range(8):
                sl = pl.ds(j * 16, 16)
                rows_v[e, sl] = rows_v[e, sl] * vv
            return c2

        lax.fori_loop(0, G, edge_body, 0)

        pltpu.sync_copy(rows_v, acc.at[dst_v.at[0]], add=True)
        return carry

    lax.fori_loop(0, cnt, group_body, 0)
    plsc.subcore_barrier()

    pltpu.sync_copy(acc.at[pl.ds(r0, RPT)], out_h.at[c].at[pl.ds(r0, RPT)])

    @pl.when(s == 15)
    def _():
        pltpu.sync_copy(acc.at[pl.ds(16 * RPT, 16)],
                        out_h.at[c].at[pl.ds(16 * RPT, 16)])


_BLK = 2000
_NBLK = N // _BLK


def _tc1_body(f_ref, x0_ref, x1_ref, w1_ref, w2_ref, b1_ref, b2_ref,
              y_ref, s_ref, q_ref):
    x = x0_ref[...] + x1_ref[...]
    f = f_ref[...]
    y = jnp.dot(f + x, w1_ref[...], preferred_element_type=jnp.float32)
    y = y + jnp.dot(f * x, w2_ref[...], preferred_element_type=jnp.float32)
    y = y + b1_ref[...] + b2_ref[...]
    y_ref[...] = y

    @pl.when(pl.program_id(0) == 0)
    def _():
        s_ref[...] = jnp.zeros_like(s_ref)
        q_ref[...] = jnp.zeros_like(q_ref)

    s_ref[...] += jnp.sum(y, axis=0, keepdims=True)
    q_ref[...] += jnp.sum(y * y, axis=0, keepdims=True)


def _tc2_body(y_ref, s_ref, q_ref, g_ref, bt_ref, o_ref):
    mean = s_ref[...] * (1.0 / N)
    var = q_ref[...] * (1.0 / N) - mean * mean
    scale = lax.rsqrt(var + 1e-5) * g_ref[...]
    o_ref[...] = (y_ref[...] - mean) * scale + bt_ref[...]


def kernel(features, edge_vals, W1, b1, W2, b2, gamma, beta, edge_src, edge_dst):
    src2d = edge_src.reshape(NG, G)
    dst2d = edge_dst.reshape(NG, G)

    xp = _sc_scatter(features, src2d, dst2d, edge_vals)

    row_spec = pl.BlockSpec((_BLK, D), lambda i: (i, 0))
    full_spec = pl.BlockSpec((D, D), lambda i: (0, 0))
    vec_spec = pl.BlockSpec((1, D), lambda i: (0, 0))

    y, ssum, ssq = pl.pallas_call(
        _tc1_body,
        grid=(_NBLK,),
        in_specs=[row_spec, row_spec, row_spec, full_spec, full_spec,
                  vec_spec, vec_spec],
        out_specs=[row_spec, vec_spec, vec_spec],
        out_shape=[
            jax.ShapeDtypeStruct((N, D), jnp.float32),
            jax.ShapeDtypeStruct((1, D), jnp.float32),
            jax.ShapeDtypeStruct((1, D), jnp.float32),
        ],
        compiler_params=pltpu.CompilerParams(
            dimension_semantics=("arbitrary",)),
    )(features, xp[0], xp[1], W1, W2, b1.reshape(1, D), b2.reshape(1, D))

    out = pl.pallas_call(
        _tc2_body,
        grid=(_NBLK,),
        in_specs=[row_spec, vec_spec, vec_spec, vec_spec, vec_spec],
        out_specs=row_spec,
        out_shape=jax.ShapeDtypeStruct((N, D), jnp.float32),
        compiler_params=pltpu.CompilerParams(
            dimension_semantics=("arbitrary",)),
    )(y, ssum, ssq, gamma.reshape(1, D), beta.reshape(1, D))
    return out
